# baseline (speedup 1.0000x reference)
import functools

import jax
import jax.numpy as jnp
from jax import lax
from jax.experimental import pallas as pl
from jax.experimental.pallas import tpu as pltpu
from jax.experimental.pallas import tpu_sc as plsc

B, V, P, DV, DP, H = 1024, 200, 256, 64, 32, 512
KA = V * DV
KP = P * DP
BK = 512
NKA = KA // BK
NKP = KP // BK
NEG = -1e8
VPAD = 256

NW = 32
BPW = B // NW


def _sc_gather(table, sel):
    @functools.partial(
        pl.kernel,
        out_type=jax.ShapeDtypeStruct((B, DV), jnp.float32),
        mesh=plsc.VectorSubcoreMesh(core_axis_name="c", subcore_axis_name="s"),
        scratch_types=[
            pltpu.VMEM((BPW,), jnp.int32),
            pltpu.VMEM((BPW, DV), jnp.float32),
            pltpu.SemaphoreType.DMA,
        ],
        compiler_params=pltpu.CompilerParams(use_tc_tiling_on_sc=False),
    )
    def k(table_hbm, sel_hbm, out_hbm, idx_v, rows_v, sem):
        wid = lax.axis_index("s") * 2 + lax.axis_index("c")
        base = wid * BPW
        pltpu.sync_copy(sel_hbm.at[pl.ds(base, BPW)], idx_v)
        for c in range(BPW // 16):
            off = c * 16
            rows = lax.iota(jnp.int32, 16) + (base + off)
            idx_v[pl.ds(off, 16)] = idx_v[pl.ds(off, 16)] + rows * V
        pltpu.async_copy(table_hbm.at[idx_v], rows_v, sem).wait()
        pltpu.sync_copy(rows_v, out_hbm.at[pl.ds(base, BPW)])

    return k(table, sel)


def _mm_body(avm_ref, pm_ref, w1_ref, w1pb_ref, h_ref, p2_ref):
    k = pl.program_id(0)

    @pl.when(k == 0)
    def _():
        h_ref[...] = jnp.zeros_like(h_ref)
        p2_ref[...] = jnp.zeros_like(p2_ref)

    w1 = w1_ref[...].astype(jnp.bfloat16)

    @pl.when(k < NKA)
    def _():
        h_ref[...] += jnp.dot(avm_ref[...].astype(jnp.bfloat16), w1,
                              preferred_element_type=jnp.float32)

    @pl.when(k >= NKA)
    def _():
        x = pm_ref[...].astype(jnp.bfloat16)
        h_ref[...] += jnp.dot(x, w1, preferred_element_type=jnp.float32)
        p2_ref[...] += jnp.dot(x, w1pb_ref[...].astype(jnp.bfloat16),
                               preferred_element_type=jnp.float32)


def _tc_matmul(avm_flat, pm_flat, W1, W1pb):
    return pl.pallas_call(
        _mm_body,
        grid=(NKA + NKP,),
        in_specs=[
            pl.BlockSpec((B, BK), lambda k: (0, jnp.minimum(k, NKA - 1))),
            pl.BlockSpec((B, BK), lambda k: (0, jnp.clip(k - NKA, 0, NKP - 1))),
            pl.BlockSpec((BK, H), lambda k: (k, 0)),
            pl.BlockSpec((BK, H), lambda k: (jnp.clip(k - NKA, 0, NKP - 1), 0)),
        ],
        out_specs=[
            pl.BlockSpec((B, H), lambda k: (0, 0)),
            pl.BlockSpec((B, H), lambda k: (0, 0)),
        ],
        out_shape=[jax.ShapeDtypeStruct((B, H), jnp.float32)] * 2,
        compiler_params=pltpu.CompilerParams(
            dimension_semantics=("arbitrary",)),
    )(avm_flat, pm_flat, W1, W1pb)


def _masked_cat(logits, mask, cols, sel):
    ml = jnp.where(mask, NEG, logits)
    m = jnp.max(ml, axis=1, keepdims=True)
    e = jnp.exp(ml - m)
    s = jnp.sum(e, axis=1, keepdims=True)
    lse = jnp.log(s) + m
    logp = ml - lse
    p = e / s
    ent = -jnp.sum(jnp.where(mask, 0.0, p * logp), axis=1, keepdims=True)
    lp = jnp.sum(jnp.where(cols == sel, logp, 0.0), axis=1, keepdims=True)
    return lp, ent


def _head_body(h_ref, p2_ref, vmsel_ref, w1pa_ref, b1_ref, b1p_ref,
               wl_ref, bl_ref, wlp_ref, blp_ref,
               nvms_ref, selvm_ref, selpm_ref, pmmask_ref,
               lp_ref, ent_ref, cr_ref):
    h = jnp.tanh(h_ref[...] + b1_ref[...])
    vm_full = jnp.dot(h, wl_ref[...],
                      preferred_element_type=jnp.float32) + bl_ref[...]
    critic = vm_full[:, V:V + 1]
    cols = lax.broadcasted_iota(jnp.int32, (B, VPAD), 1)
    maskv = cols >= nvms_ref[...]
    lpv, entv = _masked_cat(vm_full, maskv, cols, selvm_ref[...])

    hp = jnp.tanh(p2_ref[...]
                  + jnp.dot(vmsel_ref[...], w1pa_ref[...],
                            preferred_element_type=jnp.float32)
                  + b1p_ref[...])
    pm_logits = jnp.dot(hp, wlp_ref[...],
                        preferred_element_type=jnp.float32) + blp_ref[...]
    colsp = lax.broadcasted_iota(jnp.int32, (B, P), 1)
    lpp, entp = _masked_cat(pm_logits, pmmask_ref[...], colsp, selpm_ref[...])

    lp_ref[...] = lpv + lpp
    ent_ref[...] = entv + entp
    cr_ref[...] = critic


def _tc_head(h_pre, p2, vm_sel, W1pa, b1r, b1pr, wl_pad, bl_pad, Wlp, blpr,
             nvms, selvm, selpm, pm_mask):
    return pl.pallas_call(
        _head_body,
        out_shape=[jax.ShapeDtypeStruct((B, 1), jnp.float32)] * 3,
    )(h_pre, p2, vm_sel, W1pa, b1r, b1pr, wl_pad, bl_pad, Wlp, blpr,
      nvms, selvm, selpm, pm_mask)


def kernel(obs_info_pm, obs_info_all_vm, obs_info_num_steps, obs_info_num_vms,
           pm_mask, selected_vm, selected_pm,
           W1, b1, Wl, bl, Wc, bc, W1p, b1p, Wlp, blp):
    avm_flat = obs_info_all_vm.reshape(B, KA)
    pm_flat = obs_info_pm.reshape(B, KP)
    table = obs_info_all_vm.reshape(B * V, DV)
    selvm32 = selected_vm.astype(jnp.int32)

    vm_sel = _sc_gather(table, selvm32)
    h_pre, p2 = _tc_matmul(avm_flat, pm_flat, W1, W1p[DV:])

    wl_pad = jnp.concatenate(
        [Wl, Wc, jnp.zeros((H, VPAD - V - 1), jnp.float32)], axis=1)
    bl_pad = jnp.concatenate(
        [bl, bc, jnp.zeros((VPAD - V - 1,), jnp.float32)]).reshape(1, VPAD)

    lp, ent, cr = _tc_head(
        h_pre, p2, vm_sel, W1p[:DV], b1.reshape(1, H), b1p.reshape(1, H),
        wl_pad, bl_pad, Wlp, blp.reshape(1, P),
        obs_info_num_vms.astype(jnp.int32).reshape(B, 1),
        selvm32.reshape(B, 1),
        selected_pm.astype(jnp.int32).reshape(B, 1),
        pm_mask)

    return (selected_vm, selected_pm, lp.reshape(B), ent.reshape(B),
            cr.reshape(B), pm_mask)


# speedup vs baseline: 1.0455x; 1.0455x over previous
import functools

import jax
import jax.numpy as jnp
from jax import lax
from jax.experimental import pallas as pl
from jax.experimental.pallas import tpu as pltpu
from jax.experimental.pallas import tpu_sc as plsc

B, V, P, DV, DP, H = 1024, 200, 256, 64, 32, 512
KA = V * DV
KP = P * DP
BK = 512
NKA = KA // BK
NKP = KP // BK
NEG = -1e8

NW = 32
BPW = B // NW


def _sc_gather_pair(table2, sel):
    @functools.partial(
        pl.kernel,
        out_type=jax.ShapeDtypeStruct((B, 2 * DV), jnp.float32),
        mesh=plsc.VectorSubcoreMesh(core_axis_name="c", subcore_axis_name="s"),
        scratch_types=[
            pltpu.VMEM((BPW,), jnp.int32),
            pltpu.VMEM((BPW, 2 * DV), jnp.float32),
            pltpu.SemaphoreType.DMA,
        ],
    )
    def k(table_hbm, sel_hbm, out_hbm, idx_v, rows_v, sem):
        wid = lax.axis_index("s") * 2 + lax.axis_index("c")
        base = wid * BPW
        pltpu.sync_copy(sel_hbm.at[pl.ds(base, BPW)], idx_v)
        for c in range(BPW // 16):
            off = c * 16
            rows = lax.iota(jnp.int32, 16) + (base + off)
            s = idx_v[pl.ds(off, 16)]
            idx_v[pl.ds(off, 16)] = rows * (V // 2) + lax.shift_right_logical(s, 1)
        pltpu.async_copy(table_hbm.at[idx_v], rows_v, sem).wait()
        pltpu.sync_copy(rows_v, out_hbm.at[pl.ds(base, BPW)])

    return k(table2, sel)


def _mm_body(avm_ref, pm_ref, w1_ref, w1pb_ref, h_ref, p2_ref):
    k = pl.program_id(0)

    @pl.when(k == 0)
    def _():
        h_ref[...] = jnp.zeros_like(h_ref)
        p2_ref[...] = jnp.zeros_like(p2_ref)

    w1 = w1_ref[...].astype(jnp.bfloat16)

    @pl.when(k < NKA)
    def _():
        h_ref[...] += jnp.dot(avm_ref[...].astype(jnp.bfloat16), w1,
                              preferred_element_type=jnp.float32)

    @pl.when(k >= NKA)
    def _():
        x = pm_ref[...].astype(jnp.bfloat16)
        h_ref[...] += jnp.dot(x, w1, preferred_element_type=jnp.float32)
        p2_ref[...] += jnp.dot(x, w1pb_ref[...].astype(jnp.bfloat16),
                               preferred_element_type=jnp.float32)


def _tc_matmul(avm_flat, pm_flat, W1, W1p):
    return pl.pallas_call(
        _mm_body,
        grid=(NKA + NKP,),
        in_specs=[
            pl.BlockSpec((B, BK), lambda k: (0, jnp.minimum(k, NKA - 1))),
            pl.BlockSpec((B, BK), lambda k: (0, jnp.clip(k - NKA, 0, NKP - 1))),
            pl.BlockSpec((BK, H), lambda k: (k, 0)),
            pl.BlockSpec(
                (pl.Element(BK), pl.Element(H)),
                lambda k: (pl.multiple_of(
                    DV + jnp.clip(k - NKA, 0, NKP - 1) * BK, DV), 0)),
        ],
        out_specs=[
            pl.BlockSpec((B, H), lambda k: (0, 0)),
            pl.BlockSpec((B, H), lambda k: (0, 0)),
        ],
        out_shape=[jax.ShapeDtypeStruct((B, H), jnp.float32)] * 2,
        compiler_params=pltpu.CompilerParams(
            dimension_semantics=("arbitrary",)),
    )(avm_flat, pm_flat, W1, W1p)


def _masked_cat(logits, mask, cols, sel):
    ml = jnp.where(mask, NEG, logits)
    m = jnp.max(ml, axis=1, keepdims=True)
    e = jnp.exp(ml - m)
    s = jnp.sum(e, axis=1, keepdims=True)
    lse = jnp.log(s) + m
    logp = ml - lse
    p = e / s
    ent = -jnp.sum(jnp.where(mask, 0.0, p * logp), axis=1, keepdims=True)
    lp = jnp.sum(jnp.where(cols == sel, logp, 0.0), axis=1, keepdims=True)
    return lp, ent


def _head_body(h_ref, p2_ref, pair_ref, w1pa_ref, b1_ref, b1p_ref,
               wl_ref, bl_ref, wc_ref, bc_ref, wlp_ref, blp_ref,
               nvms_ref, selvm_ref, selpm_ref, pmmask_ref,
               lp_ref, ent_ref, cr_ref):
    h = jnp.tanh(h_ref[...] + b1_ref[...])
    vm_logits = jnp.dot(h, wl_ref[...],
                        preferred_element_type=jnp.float32) + bl_ref[...]
    critic = jnp.dot(h, wc_ref[...],
                     preferred_element_type=jnp.float32) + bc_ref[...]
    selvm = selvm_ref[...]
    cols = lax.broadcasted_iota(jnp.int32, (B, V), 1)
    maskv = cols >= nvms_ref[...]
    lpv, entv = _masked_cat(vm_logits, maskv, cols, selvm)

    pair = pair_ref[...]
    vm_sel = jnp.where(lax.rem(selvm, 2) == 0, pair[:, :DV], pair[:, DV:])
    hp = jnp.tanh(p2_ref[...]
                  + jnp.dot(vm_sel, w1pa_ref[...],
                            preferred_element_type=jnp.float32)
                  + b1p_ref[...])
    pm_logits = jnp.dot(hp, wlp_ref[...],
                        preferred_element_type=jnp.float32) + blp_ref[...]
    colsp = lax.broadcasted_iota(jnp.int32, (B, P), 1)
    lpp, entp = _masked_cat(pm_logits, pmmask_ref[...], colsp, selpm_ref[...])

    lp_ref[...] = lpv + lpp
    ent_ref[...] = entv + entp
    cr_ref[...] = critic


def _tc_head(h_pre, p2, pair, W1p, b1r, b1pr, Wl, blr, Wc, bcr, Wlp, blpr,
             nvms, selvm, selpm, pm_mask):
    full = lambda a: pl.BlockSpec(a.shape, lambda i: (0,) * a.ndim)
    args = (h_pre, p2, pair, W1p, b1r, b1pr, Wl, blr, Wc, bcr, Wlp, blpr,
            nvms, selvm, selpm, pm_mask)
    in_specs = [full(a) for a in args]
    in_specs[3] = pl.BlockSpec((DV, H), lambda i: (0, 0))
    return pl.pallas_call(
        _head_body,
        grid=(1,),
        in_specs=in_specs,
        out_specs=[pl.BlockSpec((B, 1), lambda i: (0, 0))] * 3,
        out_shape=[jax.ShapeDtypeStruct((B, 1), jnp.float32)] * 3,
    )(*args)


def kernel(obs_info_pm, obs_info_all_vm, obs_info_num_steps, obs_info_num_vms,
           pm_mask, selected_vm, selected_pm,
           W1, b1, Wl, bl, Wc, bc, W1p, b1p, Wlp, blp):
    avm_flat = obs_info_all_vm.reshape(B, KA)
    pm_flat = obs_info_pm.reshape(B, KP)
    table2 = obs_info_all_vm.reshape(B * V // 2, 2 * DV)
    selvm32 = selected_vm.astype(jnp.int32)

    pair = _sc_gather_pair(table2, selvm32)
    h_pre, p2 = _tc_matmul(avm_flat, pm_flat, W1, W1p)

    lp, ent, cr = _tc_head(
        h_pre, p2, pair, W1p, b1.reshape(1, H), b1p.reshape(1, H),
        Wl, bl.reshape(1, V), Wc, bc.reshape(1, 1), Wlp, blp.reshape(1, P),
        obs_info_num_vms.astype(jnp.int32).reshape(B, 1),
        selvm32.reshape(B, 1),
        selected_pm.astype(jnp.int32).reshape(B, 1),
        pm_mask)

    return (selected_vm, selected_pm, lp.reshape(B), ent.reshape(B),
            cr.reshape(B), pm_mask)
